# baseline (speedup 1.0000x reference)
import jax
import jax.numpy as jnp
from jax.experimental import pallas as pl
from jax.experimental.pallas import tpu as pltpu

_BATCH = 128
_NUM_QUERIES = 1024
_QUERIES_DIM = 512
_NSEM = 8


def _body(w_hbm, o_hbm, out_sems):
    for b in range(_BATCH):
        if b >= _NSEM:
            pltpu.make_async_copy(
                w_hbm, o_hbm.at[b - _NSEM], out_sems.at[(b - _NSEM) % _NSEM]
            ).wait()
        pltpu.make_async_copy(w_hbm, o_hbm.at[b], out_sems.at[b % _NSEM]).start()
    for b in range(_BATCH - _NSEM, _BATCH):
        pltpu.make_async_copy(w_hbm, o_hbm.at[b], out_sems.at[b % _NSEM]).wait()


def kernel(queries_weight, batch_size, num_queries):
    del batch_size, num_queries
    return pl.pallas_call(
        _body,
        in_specs=[pl.BlockSpec(memory_space=pltpu.MemorySpace.HBM)],
        out_specs=pl.BlockSpec(memory_space=pltpu.MemorySpace.HBM),
        out_shape=jax.ShapeDtypeStruct(
            (_BATCH, _NUM_QUERIES, _QUERIES_DIM), queries_weight.dtype
        ),
        scratch_shapes=[
            pltpu.SemaphoreType.DMA((_NSEM,)),
        ],
    )(queries_weight)


# speedup vs baseline: 76.9702x; 76.9702x over previous
import jax
import jax.numpy as jnp
from jax import lax
from jax.experimental import pallas as pl
from jax.experimental.pallas import tpu as pltpu
from jax.experimental.pallas import tpu_sc as plsc

_BATCH = 128
_NUM_QUERIES = 1024
_QUERIES_DIM = 512
_NC = 2
_NS = 16
_NW = _NC * _NS
_NGROUPS = 8
_WPG = _NW // _NGROUPS
_STRIPE = _NUM_QUERIES // _NGROUPS
_BPW = _BATCH // _WPG


def _sc_body(table_hbm, out_hbm, rows_v, sem):
    wid = lax.axis_index("s") * _NC + lax.axis_index("c")
    g = wid // _WPG
    j = wid % _WPG
    row0 = g * _STRIPE
    pltpu.sync_copy(table_hbm.at[pl.ds(row0, _STRIPE)], rows_v)
    copies = []
    for i in range(_BPW):
        b = j * _BPW + i
        copies.append(
            pltpu.async_copy(
                rows_v,
                out_hbm.at[pl.ds(b * _NUM_QUERIES + row0, _STRIPE)],
                sem,
            )
        )
    for c in copies:
        c.wait()


def kernel(queries_weight, batch_size, num_queries):
    del batch_size, num_queries
    out2d = pl.kernel(
        _sc_body,
        out_type=jax.ShapeDtypeStruct(
            (_BATCH * _NUM_QUERIES, _QUERIES_DIM), jnp.float32
        ),
        mesh=plsc.VectorSubcoreMesh(core_axis_name="c", subcore_axis_name="s"),
        scratch_types=[
            pltpu.VMEM((_STRIPE, _QUERIES_DIM), jnp.float32),
            pltpu.SemaphoreType.DMA,
        ],
    )(queries_weight)
    return out2d.reshape(_BATCH, _NUM_QUERIES, _QUERIES_DIM)


# speedup vs baseline: 99.0860x; 1.2873x over previous
import jax
import jax.numpy as jnp
from jax.experimental import pallas as pl

_BATCH = 128
_NUM_QUERIES = 1024
_QUERIES_DIM = 512
_B_BLK = 2


def _broadcast_body(w_ref, o_ref):
    o_ref[...] = jnp.broadcast_to(w_ref[...][None], o_ref.shape)


def kernel(queries_weight, batch_size, num_queries):
    del batch_size, num_queries
    return pl.pallas_call(
        _broadcast_body,
        grid=(_BATCH // _B_BLK,),
        in_specs=[
            pl.BlockSpec((_NUM_QUERIES, _QUERIES_DIM), lambda i: (0, 0)),
        ],
        out_specs=pl.BlockSpec(
            (_B_BLK, _NUM_QUERIES, _QUERIES_DIM), lambda i: (i, 0, 0)
        ),
        out_shape=jax.ShapeDtypeStruct(
            (_BATCH, _NUM_QUERIES, _QUERIES_DIM), queries_weight.dtype
        ),
    )(queries_weight)
